# Initial kernel scaffold; baseline (speedup 1.0000x reference)
import jax
import jax.numpy as jnp
from jax.experimental import pallas as pl


def kernel(x, edge_index, W1, b1, W2, b2):
    raise NotImplementedError("write your pallas kernel here")


# speedup vs baseline: 4.1034x; 4.1034x over previous
import functools

import jax
import jax.numpy as jnp
from jax import lax
from jax.experimental import pallas as pl
from jax.experimental.pallas import tpu as pltpu
from jax.experimental.pallas import tpu_sc as plsc

N = 10000
E = 320000
D = 128

NC = 2
NS = 16
NW = NC * NS
EPW = E // NW
CH = 80
NCHUNK = EPW // CH
RPT = N // NS

BN = 1000
GRID = N // BN

_mesh = plsc.VectorSubcoreMesh(core_axis_name="c", subcore_axis_name="s")


@functools.partial(
    pl.kernel,
    mesh=_mesh,
    out_type=jax.ShapeDtypeStruct((NC * 2 * N,), jnp.float32),
    scratch_types=[
        pltpu.VMEM_SHARED((2 * N,), jnp.float32),
        pltpu.VMEM((CH,), jnp.int32),
        pltpu.VMEM((CH,), jnp.int32),
        pltpu.VMEM((CH,), jnp.float32),
        pltpu.VMEM((2000,), jnp.float32),
    ],
)
def _deg_kernel(src_hbm, dstoff_hbm, out_hbm,
                deg_sp, src_v, dst_v, ones_v, stage_v):
    c = lax.axis_index("c")
    s = lax.axis_index("s")
    wid = c * NS + s

    @pl.when(s < 10)
    def _():
        def zb(i, carry):
            stage_v[pl.ds(i * 16, 16)] = jnp.zeros((16,), jnp.float32)
            return carry
        lax.fori_loop(0, 2000 // 16, zb, 0)
        off = pl.multiple_of(s * 2000, 8)
        pltpu.sync_copy(stage_v, deg_sp.at[pl.ds(off, 2000)])

    for j in range(CH // 16):
        ones_v[pl.ds(j * 16, 16)] = jnp.full((16,), 1.0, jnp.float32)

    plsc.subcore_barrier()

    base = wid * EPW

    def body(i, carry):
        off = pl.multiple_of(base + i * CH, 8)
        pltpu.sync_copy(src_hbm.at[pl.ds(off, CH)], src_v)
        pltpu.sync_copy(dstoff_hbm.at[pl.ds(off, CH)], dst_v)
        pltpu.sync_copy(ones_v, deg_sp.at[src_v], add=True)
        pltpu.sync_copy(ones_v, deg_sp.at[dst_v], add=True)
        return carry

    lax.fori_loop(0, NCHUNK, body, 0)
    plsc.subcore_barrier()

    @pl.when(s < 10)
    def _():
        off = pl.multiple_of(s * 2000, 8)
        pltpu.sync_copy(deg_sp.at[pl.ds(off, 2000)], stage_v)
        pltpu.sync_copy(stage_v, out_hbm.at[pl.ds(c * 2 * N + off, 2000)])


@functools.partial(
    pl.kernel,
    mesh=_mesh,
    out_type=jax.ShapeDtypeStruct((NC, N, D), jnp.float32),
    scratch_types=[
        pltpu.VMEM_SHARED((N, D), jnp.float32),
        pltpu.VMEM((CH,), jnp.int32),
        pltpu.VMEM((CH,), jnp.int32),
        pltpu.VMEM((CH, D), jnp.float32),
        pltpu.VMEM((200, D), jnp.float32),
        pltpu.SemaphoreType.DMA,
    ],
)
def _seg_kernel(xs_hbm, src_hbm, dst_hbm, out_hbm,
                agg_sp, src_v, dst_v, rows_v, stage_v, sem):
    c = lax.axis_index("c")
    s = lax.axis_index("s")
    wid = c * NS + s

    @pl.when(s < 10)
    def _():
        def zb(i, carry):
            for j in range(D // 16):
                stage_v[i, pl.ds(j * 16, 16)] = jnp.zeros((16,), jnp.float32)
            return carry
        lax.fori_loop(0, 200, zb, 0)
        roff = pl.multiple_of(s * 1000, 8)
        for k in range(5):
            pltpu.sync_copy(stage_v, agg_sp.at[pl.ds(roff + k * 200, 200)])

    plsc.subcore_barrier()

    base = wid * EPW

    def body(i, carry):
        off = pl.multiple_of(base + i * CH, 8)
        pltpu.sync_copy(src_hbm.at[pl.ds(off, CH)], src_v)
        pltpu.sync_copy(dst_hbm.at[pl.ds(off, CH)], dst_v)
        pltpu.async_copy(xs_hbm.at[src_v], rows_v, sem).wait()
        pltpu.sync_copy(rows_v, agg_sp.at[dst_v], add=True)
        return carry

    lax.fori_loop(0, NCHUNK, body, 0)
    plsc.subcore_barrier()

    @pl.when(s < 10)
    def _():
        roff = pl.multiple_of(s * 1000, 8)
        for k in range(5):
            pltpu.sync_copy(agg_sp.at[pl.ds(roff + k * 200, 200)], stage_v)
            pltpu.sync_copy(stage_v, out_hbm.at[c, pl.ds(roff + k * 200, 200)])


def _prep_body(deg_ref, x_ref, xs_ref, ns_ref, nd_ref):
    d = deg_ref[...]
    deg_o = d[0, 0] + d[1, 0]
    deg_i = d[0, 1] + d[1, 1]
    ns = lax.rsqrt(jnp.maximum(deg_o, 1.0))
    nd = lax.rsqrt(jnp.maximum(deg_i, 1.0))
    xs_ref[...] = x_ref[...] * ns
    ns_ref[...] = ns
    nd_ref[...] = nd


_prep_call = pl.pallas_call(
    _prep_body,
    grid=(GRID,),
    in_specs=[
        pl.BlockSpec((NC, 2, BN, 1), lambda i: (0, 0, i, 0)),
        pl.BlockSpec((BN, D), lambda i: (i, 0)),
    ],
    out_specs=[
        pl.BlockSpec((BN, D), lambda i: (i, 0)),
        pl.BlockSpec((BN, 1), lambda i: (i, 0)),
        pl.BlockSpec((BN, 1), lambda i: (i, 0)),
    ],
    out_shape=[
        jax.ShapeDtypeStruct((N, D), jnp.float32),
        jax.ShapeDtypeStruct((N, 1), jnp.float32),
        jax.ShapeDtypeStruct((N, 1), jnp.float32),
    ],
)


def _mid_body(p_ref, nd_ref, ns_ref, w_ref, b_ref, o_ref):
    agg = (p_ref[0] + p_ref[1]) * nd_ref[...]
    z = jnp.dot(agg, w_ref[...], preferred_element_type=jnp.float32)
    z = jnp.maximum(z + b_ref[...], 0.0)
    o_ref[...] = z * ns_ref[...]


_mid_call = pl.pallas_call(
    _mid_body,
    grid=(GRID,),
    in_specs=[
        pl.BlockSpec((NC, BN, D), lambda i: (0, i, 0)),
        pl.BlockSpec((BN, 1), lambda i: (i, 0)),
        pl.BlockSpec((BN, 1), lambda i: (i, 0)),
        pl.BlockSpec((D, D), lambda i: (0, 0)),
        pl.BlockSpec((1, D), lambda i: (0, 0)),
    ],
    out_specs=pl.BlockSpec((BN, D), lambda i: (i, 0)),
    out_shape=jax.ShapeDtypeStruct((N, D), jnp.float32),
)


def _fin_body(p_ref, nd_ref, w_ref, b_ref, o_ref):
    agg = (p_ref[0] + p_ref[1]) * nd_ref[...]
    z = jnp.dot(agg, w_ref[...], preferred_element_type=jnp.float32)
    o_ref[...] = jnp.maximum(z + b_ref[...], 0.0)


_fin_call = pl.pallas_call(
    _fin_body,
    grid=(GRID,),
    in_specs=[
        pl.BlockSpec((NC, BN, D), lambda i: (0, i, 0)),
        pl.BlockSpec((BN, 1), lambda i: (i, 0)),
        pl.BlockSpec((D, D), lambda i: (0, 0)),
        pl.BlockSpec((1, D), lambda i: (0, 0)),
    ],
    out_specs=pl.BlockSpec((BN, D), lambda i: (i, 0)),
    out_shape=jax.ShapeDtypeStruct((N, D), jnp.float32),
)


def kernel(x, edge_index, W1, b1, W2, b2):
    src = edge_index[0]
    dst = edge_index[1]

    degs = _deg_kernel(src, dst + N)
    xs, ns, nd = _prep_call(degs.reshape(NC, 2, N, 1), x)

    p1 = _seg_kernel(xs, src, dst)
    zs = _mid_call(p1, nd, ns, W1, b1.reshape(1, D))

    p2 = _seg_kernel(zs, src, dst)
    out = _fin_call(p2, nd, W2, b2.reshape(1, D))
    return out
